# baseline (speedup 1.0000x reference)
import functools

import jax
import jax.numpy as jnp
from jax import lax
from jax.experimental import pallas as pl
from jax.experimental.pallas import tpu as pltpu
from jax.experimental.pallas import tpu_sc as plsc

_VOCAB = 1000000
_D = 64
_B = 16384
_LANES = 16

_NC = 2
_NS = 16
_NW = _NC * _NS

_CH = 128
_NCHUNK = _B // (_NW * _CH)


def _make_sc_call():
    mesh = plsc.VectorSubcoreMesh(core_axis_name="c", subcore_axis_name="s")

    @functools.partial(
        pl.kernel,
        out_type=jax.ShapeDtypeStruct((_B, 3 * _D), jnp.float32),
        mesh=mesh,
        compiler_params=pltpu.CompilerParams(use_tc_tiling_on_sc=False),
        scratch_types=[
            pltpu.VMEM((9, _CH), jnp.int32),
            pltpu.VMEM((9, _CH, _D), jnp.float32),
            pltpu.VMEM((_CH, _D), jnp.float32),
            pltpu.VMEM((_CH, _D), jnp.float32),
            pltpu.SemaphoreType.DMA,
        ],
    )
    def call(idx_hbm, table_hbm, out_hbm, idx_v, rows_v, acc1_v, acc2_v, sem):
        wid = lax.axis_index("s") * _NC + lax.axis_index("c")
        quarter = jnp.float32(0.25)

        for c in range(_NCHUNK):
            g = wid * _NCHUNK + c
            pltpu.sync_copy(idx_hbm.at[g], idx_v)
            copies = [
                pltpu.async_copy(table_hbm.at[idx_v.at[r]], rows_v.at[r], sem)
                for r in range(9)
            ]
            for cp in copies:
                cp.wait()
            pltpu.sync_copy(
                rows_v.at[0], out_hbm.at[pl.ds(g * _CH, _CH), pl.ds(0, _D)]
            )

            def body(i, _):
                for k in range(_D // _LANES):
                    s = pl.ds(k * _LANES, _LANES)
                    a1 = (
                        rows_v[1, i, s] + rows_v[2, i, s]
                        + rows_v[3, i, s] + rows_v[4, i, s]
                    ) * quarter
                    acc1_v[i, s] = a1
                    a2 = (
                        rows_v[5, i, s] + rows_v[6, i, s]
                        + rows_v[7, i, s] + rows_v[8, i, s]
                    ) * quarter
                    acc2_v[i, s] = a2
                return 0

            lax.fori_loop(0, _CH, body, 0, unroll=False)

            pltpu.sync_copy(
                acc1_v, out_hbm.at[pl.ds(g * _CH, _CH), pl.ds(_D, _D)]
            )
            pltpu.sync_copy(
                acc2_v, out_hbm.at[pl.ds(g * _CH, _CH), pl.ds(2 * _D, _D)]
            )

    return call


_sc_call = _make_sc_call()


@jax.jit
def kernel(opcode_idx, op1_idx, op2_idx, table):
    idx_all = jnp.concatenate(
        [
            opcode_idx[None, :].astype(jnp.int32),
            op1_idx.T.astype(jnp.int32),
            op2_idx.T.astype(jnp.int32),
        ],
        axis=0,
    )
    idx_blocks = (
        idx_all.reshape(9, _NW * _NCHUNK, _CH).transpose(1, 0, 2)
    )
    return _sc_call(idx_blocks, table)
